# baseline (speedup 1.0000x reference)
import functools

import jax
import jax.numpy as jnp
from jax import lax
from jax.experimental import pallas as pl
from jax.experimental.pallas import tpu as pltpu
from jax.experimental.pallas import tpu_sc as plsc

NA = 65536
NB = 64
CE = 64
HP = 102
NROWS = NB * HP * HP

_info = plsc.get_sparse_core_info()
NC = _info.num_cores
NS = _info.num_subcores
L = _info.num_lanes
NW = NC * NS
APW = NA // NW
CHUNK = 128
NCHUNK = APW // CHUNK
GPC = CHUNK // L

_mesh = plsc.VectorSubcoreMesh(core_axis_name="c", subcore_axis_name="s")


@functools.partial(
    pl.kernel,
    out_type=jax.ShapeDtypeStruct((NA // 2, 2 * CE), jnp.float32),
    mesh=_mesh,
    scratch_types=[
        pltpu.VMEM((APW,), jnp.int32),
        pltpu.VMEM((APW,), jnp.float32),
        pltpu.VMEM((APW,), jnp.float32),
        pltpu.VMEM((NCHUNK, CHUNK), jnp.int32),
        pltpu.VMEM((NCHUNK, CHUNK), jnp.int32),
        pltpu.VMEM((NCHUNK, CHUNK), jnp.int32),
        pltpu.VMEM((NCHUNK, CHUNK), jnp.int32),
        pltpu.VMEM((NCHUNK, CHUNK), jnp.int32),
        pltpu.VMEM((NCHUNK, CHUNK), jnp.int32),
        pltpu.VMEM((NCHUNK, CHUNK), jnp.int32),
        pltpu.VMEM((NCHUNK, CHUNK), jnp.int32),
        pltpu.VMEM((NCHUNK, CHUNK), jnp.float32),
        pltpu.VMEM((NCHUNK, CHUNK), jnp.float32),
        pltpu.VMEM((NCHUNK, CHUNK), jnp.float32),
        pltpu.VMEM((NCHUNK, CHUNK), jnp.float32),
        pltpu.VMEM((CHUNK, 2 * CE), jnp.float32),
        pltpu.VMEM((CHUNK, 2 * CE), jnp.float32),
        pltpu.VMEM((CHUNK, 2 * CE), jnp.float32),
        pltpu.VMEM((CHUNK, 2 * CE), jnp.float32),
        pltpu.VMEM((CHUNK // 2, 2 * CE), jnp.float32),
        pltpu.SemaphoreType.DMA,
    ],
)
def _sc_interp(t_hbm, eidx_hbm, pmx_hbm, pmy_hbm,
               out_hbm,
               eidx_v, pmx_v, pmy_v,
               i11_v, i21_v, i12_v, i22_v,
               p11_v, p21_v, p12_v, p22_v,
               w11_v, w21_v, w12_v, w22_v,
               q11_v, q21_v, q12_v, q22_v, outb_v, sem):
    wid = lax.axis_index("s") * NC + lax.axis_index("c")
    base = wid * APW

    pltpu.sync_copy(eidx_hbm.at[pl.ds(base, APW)], eidx_v)
    pltpu.sync_copy(pmx_hbm.at[pl.ds(base, APW)], pmx_v)
    pltpu.sync_copy(pmy_hbm.at[pl.ds(base, APW)], pmy_v)

    def axis_cells(pm):
        c = jnp.minimum(jnp.maximum(pm, 0.0), 101.0)
        lo_i = c.astype(jnp.int32)
        lo_f = lo_i.astype(jnp.float32)
        hi_i = lo_i + jnp.where(c > lo_f, 1, 0)
        hi_f = hi_i.astype(jnp.float32)
        return lo_i, lo_f, hi_i, hi_f

    def index_chunk(ci, _):
        for g in range(GPC):
            off = ci * CHUNK + g * L
            col = g * L
            pmx = pmx_v[pl.ds(off, L)]
            pmy = pmy_v[pl.ds(off, L)]
            ev = eidx_v[pl.ds(off, L)]
            x1i, x1f, x2i, x2f = axis_cells(pmx)
            y1i, y1f, y2i, y2f = axis_cells(pmy)
            dx2 = x2f - pmx
            dx1 = pmx - x1f
            dy2 = y2f - pmy
            dy1 = pmy - y1f
            w11_v[ci, pl.ds(col, L)] = dx2 * dy2
            w21_v[ci, pl.ds(col, L)] = dx1 * dy2
            w12_v[ci, pl.ds(col, L)] = dx2 * dy1
            w22_v[ci, pl.ds(col, L)] = dx1 * dy1
            r11 = (ev * HP + y1i) * HP + x1i
            dxs = x2i - x1i
            dys = (y2i - y1i) * HP
            r21 = r11 + dys
            r12 = r11 + dxs
            r22 = r21 + dxs
            i11_v[ci, pl.ds(col, L)] = r11 >> 1
            i21_v[ci, pl.ds(col, L)] = r21 >> 1
            i12_v[ci, pl.ds(col, L)] = r12 >> 1
            i22_v[ci, pl.ds(col, L)] = r22 >> 1
            p11_v[ci, pl.ds(col, L)] = (r11 & 1) << 6
            p21_v[ci, pl.ds(col, L)] = (r21 & 1) << 6
            p12_v[ci, pl.ds(col, L)] = (r12 & 1) << 6
            p22_v[ci, pl.ds(col, L)] = (r22 & 1) << 6
        return _

    lax.fori_loop(0, NCHUNK, index_chunk, None, unroll=False)

    def gather_chunk(ci, _):
        c11 = pltpu.async_copy(t_hbm.at[i11_v.at[ci]], q11_v, sem)
        c21 = pltpu.async_copy(t_hbm.at[i21_v.at[ci]], q21_v, sem)
        c12 = pltpu.async_copy(t_hbm.at[i12_v.at[ci]], q12_v, sem)
        c22 = pltpu.async_copy(t_hbm.at[i22_v.at[ci]], q22_v, sem)
        c11.wait()
        c21.wait()
        c12.wait()
        c22.wait()

        def combine_group(g, _):
            col = g * L
            w11 = w11_v[ci, pl.ds(col, L)]
            w21 = w21_v[ci, pl.ds(col, L)]
            w12 = w12_v[ci, pl.ds(col, L)]
            w22 = w22_v[ci, pl.ds(col, L)]
            p11 = p11_v[ci, pl.ds(col, L)]
            p21 = p21_v[ci, pl.ds(col, L)]
            p12 = p12_v[ci, pl.ds(col, L)]
            p22 = p22_v[ci, pl.ds(col, L)]
            for j in range(L):
                a = col + j
                s11 = w11[j]
                s21 = w21[j]
                s12 = w12[j]
                s22 = w22[j]
                o11 = p11[j]
                o21 = p21[j]
                o12 = p12[j]
                o22 = p22[j]
                for cc in range(CE // L):
                    cs = cc * L
                    q11 = q11_v[a, pl.ds(o11 + cs, L)]
                    q21 = q21_v[a, pl.ds(o21 + cs, L)]
                    q12 = q12_v[a, pl.ds(o12 + cs, L)]
                    q22 = q22_v[a, pl.ds(o22 + cs, L)]
                    outb_v[a // 2, pl.ds((a % 2) * CE + cs, L)] = (
                        s11 * q11 + s21 * q21 + s12 * q12 + s22 * q22)
            return _

        lax.fori_loop(0, GPC, combine_group, None, unroll=False)
        row0 = pl.multiple_of((base + ci * CHUNK) // 2, 8)
        pltpu.sync_copy(outb_v, out_hbm.at[pl.ds(row0, CHUNK // 2)])
        return _

    lax.fori_loop(0, NCHUNK, gather_chunk, None, unroll=False)


def kernel(episode_idx, positions, grid_map):
    table = jnp.pad(jnp.transpose(grid_map, (0, 2, 3, 1)),
                    ((0, 0), (1, 1), (1, 1), (0, 0)))
    table = table.reshape(NROWS // 2, 2 * CE)
    positions_map = (positions + 56.0) / 112.0 * 100.0 + 1.0
    pmx = positions_map[:, 0]
    pmy = positions_map[:, 1]
    out2 = _sc_interp(table, episode_idx, pmx, pmy)
    return out2.reshape(NA, CE), positions_map


# speedup vs baseline: 1.4332x; 1.4332x over previous
import functools

import jax
import jax.numpy as jnp
from jax import lax
from jax.experimental import pallas as pl
from jax.experimental.pallas import tpu as pltpu
from jax.experimental.pallas import tpu_sc as plsc

NA = 65536
NB = 64
CE = 64
HP = 102
WP = 104
NROWS = NB * HP * WP

_info = plsc.get_sparse_core_info()
NC = _info.num_cores
NS = _info.num_subcores
L = _info.num_lanes
NW = NC * NS
APW = NA // NW
CHUNK = 128
NCHUNK = APW // CHUNK
GPC = CHUNK // L

_mesh = plsc.VectorSubcoreMesh(core_axis_name="c", subcore_axis_name="s")


@functools.partial(
    pl.kernel,
    out_type=jax.ShapeDtypeStruct((NA // 2, 2 * CE), jnp.float32),
    mesh=_mesh,
    scratch_types=[
        pltpu.VMEM((APW,), jnp.int32),
        pltpu.VMEM((APW,), jnp.float32),
        pltpu.VMEM((APW,), jnp.float32),
        pltpu.VMEM((NCHUNK, CHUNK), jnp.int32),
        pltpu.VMEM((NCHUNK, CHUNK), jnp.int32),
        pltpu.VMEM((NCHUNK, CHUNK), jnp.int32),
        pltpu.VMEM((NCHUNK, CHUNK), jnp.int32),
        pltpu.VMEM((NCHUNK, CHUNK), jnp.float32),
        pltpu.VMEM((NCHUNK, CHUNK), jnp.float32),
        pltpu.VMEM((NCHUNK, CHUNK), jnp.float32),
        pltpu.VMEM((NCHUNK, CHUNK), jnp.float32),
        pltpu.VMEM((CHUNK, 2 * CE), jnp.float32),
        pltpu.VMEM((CHUNK, 2 * CE), jnp.float32),
        pltpu.VMEM((CHUNK, 2 * CE), jnp.float32),
        pltpu.VMEM((CHUNK, 2 * CE), jnp.float32),
        pltpu.VMEM((CHUNK // 2, 2 * CE), jnp.float32),
        pltpu.SemaphoreType.DMA,
    ],
)
def _sc_interp(t_hbm, eidx_hbm, pmx_hbm, pmy_hbm,
               out_hbm,
               eidx_v, pmx_v, pmy_v,
               i11_v, i21_v, i12_v, i22_v,
               w11_v, w21_v, w12_v, w22_v,
               q11_v, q21_v, q12_v, q22_v, outb_v, sem):
    wid = lax.axis_index("s") * NC + lax.axis_index("c")
    base = wid * APW

    pltpu.sync_copy(eidx_hbm.at[pl.ds(base, APW)], eidx_v)
    pltpu.sync_copy(pmx_hbm.at[pl.ds(base, APW)], pmx_v)
    pltpu.sync_copy(pmy_hbm.at[pl.ds(base, APW)], pmy_v)

    def axis_cells(pm):
        c = jnp.minimum(jnp.maximum(pm, 0.0), 101.0)
        lo_i = c.astype(jnp.int32)
        lo_f = lo_i.astype(jnp.float32)
        hi_i = lo_i + jnp.where(c > lo_f, 1, 0)
        hi_f = hi_i.astype(jnp.float32)
        return lo_i, lo_f, hi_i, hi_f

    def index_chunk(ci, _):
        for g in range(GPC):
            off = ci * CHUNK + g * L
            col = g * L
            pmx = pmx_v[pl.ds(off, L)]
            pmy = pmy_v[pl.ds(off, L)]
            ev = eidx_v[pl.ds(off, L)]
            x1i, x1f, x2i, x2f = axis_cells(pmx)
            y1i, y1f, y2i, y2f = axis_cells(pmy)
            dx2 = x2f - pmx
            dx1 = pmx - x1f
            dy2 = y2f - pmy
            dy1 = pmy - y1f
            w11_v[ci, pl.ds(col, L)] = dx2 * dy2
            w21_v[ci, pl.ds(col, L)] = dx1 * dy2
            w12_v[ci, pl.ds(col, L)] = dx2 * dy1
            w22_v[ci, pl.ds(col, L)] = dx1 * dy1
            r11 = (ev * HP + y1i) * WP + x1i
            dxs = x2i - x1i
            dys = (y2i - y1i) * WP
            r21 = r11 + dys
            r12 = r11 + dxs
            r22 = r21 + dxs
            i11_v[ci, pl.ds(col, L)] = r11
            i21_v[ci, pl.ds(col, L)] = r21
            i12_v[ci, pl.ds(col, L)] = r12
            i22_v[ci, pl.ds(col, L)] = r22
        return _

    lax.fori_loop(0, NCHUNK, index_chunk, None, unroll=False)

    def gather_chunk(ci, _):
        c11 = pltpu.async_copy(t_hbm.at[i11_v.at[ci]], q11_v, sem)
        c21 = pltpu.async_copy(t_hbm.at[i21_v.at[ci]], q21_v, sem)
        c12 = pltpu.async_copy(t_hbm.at[i12_v.at[ci]], q12_v, sem)
        c22 = pltpu.async_copy(t_hbm.at[i22_v.at[ci]], q22_v, sem)
        c11.wait()
        c21.wait()
        c12.wait()
        c22.wait()

        def combine_group(g, _):
            col = g * L
            w11 = w11_v[ci, pl.ds(col, L)]
            w21 = w21_v[ci, pl.ds(col, L)]
            w12 = w12_v[ci, pl.ds(col, L)]
            w22 = w22_v[ci, pl.ds(col, L)]
            for j in range(L):
                a = col + j
                s11 = w11[j]
                s21 = w21[j]
                s12 = w12[j]
                s22 = w22[j]
                for cc in range(CE // L):
                    cs = cc * L
                    q11 = q11_v[a, pl.ds(cs, L)]
                    q21 = q21_v[a, pl.ds(cs, L)]
                    q12 = q12_v[a, pl.ds(cs, L)]
                    q22 = q22_v[a, pl.ds(cs, L)]
                    outb_v[a // 2, pl.ds((a % 2) * CE + cs, L)] = (
                        s11 * q11 + s21 * q21 + s12 * q12 + s22 * q22)
            return _

        lax.fori_loop(0, GPC, combine_group, None, unroll=False)
        row0 = pl.multiple_of((base + ci * CHUNK) // 2, 8)
        pltpu.sync_copy(outb_v, out_hbm.at[pl.ds(row0, CHUNK // 2)])
        return _

    lax.fori_loop(0, NCHUNK, gather_chunk, None, unroll=False)


def kernel(episode_idx, positions, grid_map):
    table = jnp.pad(jnp.transpose(grid_map, (0, 2, 3, 1)),
                    ((0, 0), (1, 1), (1, 3), (0, CE)))
    table = table.reshape(NROWS, 2 * CE)
    positions_map = (positions + 56.0) / 112.0 * 100.0 + 1.0
    pmx = positions_map[:, 0]
    pmy = positions_map[:, 1]
    out2 = _sc_interp(table, episode_idx, pmx, pmy)
    return out2.reshape(NA, CE), positions_map


# speedup vs baseline: 1.4431x; 1.0069x over previous
import functools

import jax
import jax.numpy as jnp
from jax import lax
from jax.experimental import pallas as pl
from jax.experimental.pallas import tpu as pltpu
from jax.experimental.pallas import tpu_sc as plsc

NA = 65536
NB = 64
CE = 64
HP = 102
WP = 104
NROWS = NB * HP * WP

_info = plsc.get_sparse_core_info()
NC = _info.num_cores
NS = _info.num_subcores
L = _info.num_lanes
NW = NC * NS
APW = NA // NW
CHUNK = 64
NCHUNK = APW // CHUNK
GPC = CHUNK // L

_mesh = plsc.VectorSubcoreMesh(core_axis_name="c", subcore_axis_name="s")


@functools.partial(
    pl.kernel,
    out_type=jax.ShapeDtypeStruct((NA // 2, 2 * CE), jnp.float32),
    mesh=_mesh,
    scratch_types=[
        pltpu.VMEM((APW,), jnp.int32),
        pltpu.VMEM((APW,), jnp.float32),
        pltpu.VMEM((APW,), jnp.float32),
        pltpu.VMEM((NCHUNK, CHUNK), jnp.int32),
        pltpu.VMEM((NCHUNK, CHUNK), jnp.int32),
        pltpu.VMEM((NCHUNK, CHUNK), jnp.int32),
        pltpu.VMEM((NCHUNK, CHUNK), jnp.int32),
        pltpu.VMEM((NCHUNK, CHUNK), jnp.float32),
        pltpu.VMEM((NCHUNK, CHUNK), jnp.float32),
        pltpu.VMEM((NCHUNK, CHUNK), jnp.float32),
        pltpu.VMEM((NCHUNK, CHUNK), jnp.float32),
        pltpu.VMEM((2, CHUNK, 2 * CE), jnp.float32),
        pltpu.VMEM((2, CHUNK, 2 * CE), jnp.float32),
        pltpu.VMEM((2, CHUNK, 2 * CE), jnp.float32),
        pltpu.VMEM((2, CHUNK, 2 * CE), jnp.float32),
        pltpu.VMEM((2, CHUNK // 2, 2 * CE), jnp.float32),
        pltpu.SemaphoreType.DMA,
        pltpu.SemaphoreType.DMA,
    ],
)
def _sc_interp(t_hbm, eidx_hbm, pmx_hbm, pmy_hbm,
               out_hbm,
               eidx_v, pmx_v, pmy_v,
               i11_v, i21_v, i12_v, i22_v,
               w11_v, w21_v, w12_v, w22_v,
               q11_v, q21_v, q12_v, q22_v, outb_v, sem, sem2):
    wid = lax.axis_index("s") * NC + lax.axis_index("c")
    base = wid * APW

    pltpu.sync_copy(eidx_hbm.at[pl.ds(base, APW)], eidx_v)
    pltpu.sync_copy(pmx_hbm.at[pl.ds(base, APW)], pmx_v)
    pltpu.sync_copy(pmy_hbm.at[pl.ds(base, APW)], pmy_v)

    def axis_cells(pm):
        c = jnp.minimum(jnp.maximum(pm, 0.0), 101.0)
        lo_i = c.astype(jnp.int32)
        lo_f = lo_i.astype(jnp.float32)
        hi_i = lo_i + jnp.where(c > lo_f, 1, 0)
        hi_f = hi_i.astype(jnp.float32)
        return lo_i, lo_f, hi_i, hi_f

    def index_chunk(ci, _):
        for g in range(GPC):
            off = ci * CHUNK + g * L
            col = g * L
            pmx = pmx_v[pl.ds(off, L)]
            pmy = pmy_v[pl.ds(off, L)]
            ev = eidx_v[pl.ds(off, L)]
            x1i, x1f, x2i, x2f = axis_cells(pmx)
            y1i, y1f, y2i, y2f = axis_cells(pmy)
            dx2 = x2f - pmx
            dx1 = pmx - x1f
            dy2 = y2f - pmy
            dy1 = pmy - y1f
            w11_v[ci, pl.ds(col, L)] = dx2 * dy2
            w21_v[ci, pl.ds(col, L)] = dx1 * dy2
            w12_v[ci, pl.ds(col, L)] = dx2 * dy1
            w22_v[ci, pl.ds(col, L)] = dx1 * dy1
            r11 = (ev * HP + y1i) * WP + x1i
            dxs = x2i - x1i
            dys = (y2i - y1i) * WP
            r21 = r11 + dys
            r12 = r11 + dxs
            r22 = r21 + dxs
            i11_v[ci, pl.ds(col, L)] = r11
            i21_v[ci, pl.ds(col, L)] = r21
            i12_v[ci, pl.ds(col, L)] = r12
            i22_v[ci, pl.ds(col, L)] = r22
        return _

    lax.fori_loop(0, NCHUNK, index_chunk, None, unroll=False)

    def combine_chunk(ci, hf):
        def combine_group(g, _):
            col = g * L
            w11 = w11_v[ci, pl.ds(col, L)]
            w21 = w21_v[ci, pl.ds(col, L)]
            w12 = w12_v[ci, pl.ds(col, L)]
            w22 = w22_v[ci, pl.ds(col, L)]
            for j in range(L):
                a = col + j
                s11 = w11[j]
                s21 = w21[j]
                s12 = w12[j]
                s22 = w22[j]
                for cc in range(CE // L):
                    cs = cc * L
                    q11 = q11_v[hf, a, pl.ds(cs, L)]
                    q21 = q21_v[hf, a, pl.ds(cs, L)]
                    q12 = q12_v[hf, a, pl.ds(cs, L)]
                    q22 = q22_v[hf, a, pl.ds(cs, L)]
                    outb_v[hf, a // 2, pl.ds((a % 2) * CE + cs, L)] = (
                        s11 * q11 + s21 * q21 + s12 * q12 + s22 * q22)
            return _

        lax.fori_loop(0, GPC, combine_group, None, unroll=False)
        row0 = pl.multiple_of((base + ci * CHUNK) // 2, 8)
        pltpu.sync_copy(outb_v.at[hf], out_hbm.at[pl.ds(row0, CHUNK // 2)])

    def gather_pair(ci2, _):
        c0 = ci2 * 2
        c1 = c0 + 1
        cpa = [
            pltpu.async_copy(t_hbm.at[i11_v.at[c0]], q11_v.at[0], sem),
            pltpu.async_copy(t_hbm.at[i21_v.at[c0]], q21_v.at[0], sem),
            pltpu.async_copy(t_hbm.at[i12_v.at[c0]], q12_v.at[0], sem),
            pltpu.async_copy(t_hbm.at[i22_v.at[c0]], q22_v.at[0], sem),
        ]
        cpb = [
            pltpu.async_copy(t_hbm.at[i11_v.at[c1]], q11_v.at[1], sem2),
            pltpu.async_copy(t_hbm.at[i21_v.at[c1]], q21_v.at[1], sem2),
            pltpu.async_copy(t_hbm.at[i12_v.at[c1]], q12_v.at[1], sem2),
            pltpu.async_copy(t_hbm.at[i22_v.at[c1]], q22_v.at[1], sem2),
        ]
        for cp in cpa:
            cp.wait()
        combine_chunk(c0, 0)
        for cp in cpb:
            cp.wait()
        combine_chunk(c1, 1)
        return _

    lax.fori_loop(0, NCHUNK // 2, gather_pair, None, unroll=False)


def kernel(episode_idx, positions, grid_map):
    table = jnp.pad(jnp.transpose(grid_map, (0, 2, 3, 1)),
                    ((0, 0), (1, 1), (1, 3), (0, CE)))
    table = table.reshape(NROWS, 2 * CE)
    positions_map = (positions + 56.0) / 112.0 * 100.0 + 1.0
    pmx = positions_map[:, 0]
    pmy = positions_map[:, 1]
    out2 = _sc_interp(table, episode_idx, pmx, pmy)
    return out2.reshape(NA, CE), positions_map
